# Initial kernel scaffold; baseline (speedup 1.0000x reference)
import functools

import jax
import jax.numpy as jnp
from jax import lax
from jax.experimental import pallas as pl
from jax.experimental.pallas import tpu as pltpu
from jax.experimental.pallas import tpu_sc as plsc

N = 10000
E = 320000
D = 128
NPAD = 10240
RPS = NPAD // 16
CH = 128
CW = 80
NW = 32
EPAD = NW * CW * CH

_mesh = plsc.VectorSubcoreMesh(core_axis_name="c", subcore_axis_name="s")


@functools.partial(
    pl.kernel,
    out_type=[
        jax.ShapeDtypeStruct((2, NPAD, 16), jnp.float32),
        jax.ShapeDtypeStruct((2, NPAD, 16), jnp.float32),
    ],
    mesh=_mesh,
    scratch_types=[
        pltpu.VMEM((CW, CH), jnp.int32),
        pltpu.VMEM((CW, CH), jnp.int32),
        pltpu.VMEM((CH, 16), jnp.float32),
        pltpu.VMEM_SHARED((NPAD, 16), jnp.float32),
        pltpu.VMEM_SHARED((NPAD, 16), jnp.float32),
    ],
)
def _deg_kernel(src_hbm, dst_hbm, ones_hbm, zeros_hbm,
                dsrc_out, ddst_out, sbuf, dbuf, obuf, dsrc_sh, ddst_sh):
    c = lax.axis_index("c")
    s = lax.axis_index("s")
    w = c * 16 + s
    pltpu.sync_copy(zeros_hbm, dsrc_sh.at[pl.ds(s * RPS, RPS)])
    pltpu.sync_copy(zeros_hbm, ddst_sh.at[pl.ds(s * RPS, RPS)])
    pltpu.sync_copy(src_hbm.at[pl.ds(w * CW, CW)], sbuf)
    pltpu.sync_copy(dst_hbm.at[pl.ds(w * CW, CW)], dbuf)
    pltpu.sync_copy(ones_hbm, obuf)
    plsc.subcore_barrier()

    @pl.loop(0, CW)
    def _(j):
        pltpu.sync_copy(obuf, dsrc_sh.at[sbuf.at[j]], add=True)
        pltpu.sync_copy(obuf, ddst_sh.at[dbuf.at[j]], add=True)

    plsc.subcore_barrier()
    pltpu.sync_copy(dsrc_sh.at[pl.ds(s * RPS, RPS)],
                    dsrc_out.at[c, pl.ds(s * RPS, RPS)])
    pltpu.sync_copy(ddst_sh.at[pl.ds(s * RPS, RPS)],
                    ddst_out.at[c, pl.ds(s * RPS, RPS)])


@functools.partial(
    pl.kernel,
    out_type=jax.ShapeDtypeStruct((2, NPAD, D), jnp.float32),
    mesh=_mesh,
    scratch_types=[
        pltpu.VMEM((CW, CH), jnp.int32),
        pltpu.VMEM((CW, CH), jnp.int32),
        pltpu.VMEM((CH, D), jnp.float32),
        pltpu.VMEM((CH, D), jnp.float32),
        pltpu.VMEM_SHARED((NPAD, D), jnp.float32),
        pltpu.SemaphoreType.DMA,
        pltpu.SemaphoreType.DMA,
    ],
)
def _agg_kernel(h_hbm, src_hbm, dst_hbm, zeros_hbm,
                acc_out, sbuf, dbuf, r0, r1, acc_sh, sem0, sem1):
    c = lax.axis_index("c")
    s = lax.axis_index("s")
    w = c * 16 + s
    pltpu.sync_copy(zeros_hbm, acc_sh.at[pl.ds(s * RPS, RPS)])
    pltpu.sync_copy(src_hbm.at[pl.ds(w * CW, CW)], sbuf)
    pltpu.sync_copy(dst_hbm.at[pl.ds(w * CW, CW)], dbuf)
    plsc.subcore_barrier()

    pltpu.async_copy(h_hbm.at[sbuf.at[0]], r0, sem0)

    @pl.loop(0, CW // 2)
    def _(t):
        g = t * 2
        pltpu.make_async_copy(h_hbm.at[sbuf.at[g]], r0, sem0).wait()
        pltpu.async_copy(h_hbm.at[sbuf.at[g + 1]], r1, sem1)
        pltpu.sync_copy(r0, acc_sh.at[dbuf.at[g]], add=True)
        pltpu.make_async_copy(h_hbm.at[sbuf.at[g + 1]], r1, sem1).wait()

        @pl.when(g + 2 < CW)
        def _():
            pltpu.async_copy(h_hbm.at[sbuf.at[g + 2]], r0, sem0)

        pltpu.sync_copy(r1, acc_sh.at[dbuf.at[g + 1]], add=True)

    plsc.subcore_barrier()
    pltpu.sync_copy(acc_sh.at[pl.ds(s * RPS, RPS)],
                    acc_out.at[c, pl.ds(s * RPS, RPS)])


def _prep_body(feat_ref, dsrc_ref, ddst_ref, h1_ref, no_ref, ni_ref):
    dsrc = dsrc_ref[0, :, 0:1] + dsrc_ref[1, :, 0:1]
    ddst = ddst_ref[0, :, 0:1] + ddst_ref[1, :, 0:1]
    no = jnp.where(dsrc > 0, lax.rsqrt(jnp.maximum(dsrc, 1.0)), 0.0)
    ni = jnp.where(ddst > 0, lax.rsqrt(jnp.maximum(ddst, 1.0)), 0.0)
    no_ref[...] = no
    ni_ref[...] = ni
    h1_ref[...] = feat_ref[...] * no


def _mid_body(acc_ref, ni_ref, no_ref, w_ref, b_ref, out_ref):
    agg = (acc_ref[0] + acc_ref[1]) * ni_ref[...]
    o = jnp.dot(agg, w_ref[...], preferred_element_type=jnp.float32) + b_ref[...]
    out_ref[...] = jnp.maximum(o, 0.0) * no_ref[...]


def _final_body(acc_ref, ni_ref, w_ref, b_ref, out_ref):
    agg = (acc_ref[0] + acc_ref[1]) * ni_ref[...]
    out_ref[...] = (
        jnp.dot(agg, w_ref[...], preferred_element_type=jnp.float32) + b_ref[...]
    )


_BP = 2048
_BF = 2000

_prep_call = pl.pallas_call(
    _prep_body,
    grid=(NPAD // _BP,),
    in_specs=[
        pl.BlockSpec((_BP, D), lambda i: (i, 0)),
        pl.BlockSpec((2, _BP, 16), lambda i: (0, i, 0)),
        pl.BlockSpec((2, _BP, 16), lambda i: (0, i, 0)),
    ],
    out_specs=[
        pl.BlockSpec((_BP, D), lambda i: (i, 0)),
        pl.BlockSpec((_BP, 1), lambda i: (i, 0)),
        pl.BlockSpec((_BP, 1), lambda i: (i, 0)),
    ],
    out_shape=[
        jax.ShapeDtypeStruct((NPAD, D), jnp.float32),
        jax.ShapeDtypeStruct((NPAD, 1), jnp.float32),
        jax.ShapeDtypeStruct((NPAD, 1), jnp.float32),
    ],
)

_mid_call = pl.pallas_call(
    _mid_body,
    grid=(NPAD // _BP,),
    in_specs=[
        pl.BlockSpec((2, _BP, D), lambda i: (0, i, 0)),
        pl.BlockSpec((_BP, 1), lambda i: (i, 0)),
        pl.BlockSpec((_BP, 1), lambda i: (i, 0)),
        pl.BlockSpec((D, D), lambda i: (0, 0)),
        pl.BlockSpec((1, D), lambda i: (0, 0)),
    ],
    out_specs=pl.BlockSpec((_BP, D), lambda i: (i, 0)),
    out_shape=jax.ShapeDtypeStruct((NPAD, D), jnp.float32),
)

_final_call = pl.pallas_call(
    _final_body,
    grid=(N // _BF,),
    in_specs=[
        pl.BlockSpec((2, _BF, D), lambda i: (0, i, 0)),
        pl.BlockSpec((_BF, 1), lambda i: (i, 0)),
        pl.BlockSpec((D, D), lambda i: (0, 0)),
        pl.BlockSpec((1, D), lambda i: (0, 0)),
    ],
    out_specs=pl.BlockSpec((_BF, D), lambda i: (i, 0)),
    out_shape=jax.ShapeDtypeStruct((N, D), jnp.float32),
)


def kernel(features, edge_index, W1, b1, W2, b2):
    edge_index = edge_index.astype(jnp.int32)
    pad = jnp.full((EPAD - E,), N, jnp.int32)
    src2d = jnp.concatenate([edge_index[0], pad]).reshape(NW * CW, CH)
    dst2d = jnp.concatenate([edge_index[1], pad]).reshape(NW * CW, CH)
    feat_pad = jnp.pad(features, ((0, NPAD - N), (0, 0)))
    ones_r = jnp.ones((CH, 16), jnp.float32)
    zeros16 = jnp.zeros((RPS, 16), jnp.float32)
    zeros128 = jnp.zeros((RPS, D), jnp.float32)

    dsrc_p, ddst_p = _deg_kernel(src2d, dst2d, ones_r, zeros16)
    h1, norm_out, norm_in = _prep_call(feat_pad, dsrc_p, ddst_p)
    acc1 = _agg_kernel(h1, src2d, dst2d, zeros128)
    h2 = _mid_call(acc1, norm_in, norm_out, W1, b1.reshape(1, D))
    acc2 = _agg_kernel(h2, src2d, dst2d, zeros128)
    return _final_call(acc2, norm_in, W2, b2.reshape(1, D))


# speedup vs baseline: 3.9294x; 3.9294x over previous
import dataclasses
import functools

import jax
import jax.numpy as jnp
from jax import lax
from jax.experimental import pallas as pl
from jax.experimental.pallas import tpu as pltpu
from jax.experimental.pallas import tpu_sc as plsc

N = 10000
E = 320000
D = 128
NPAD = 10240
RPS = NPAD // 16
CH = 128
CW = 80
SL = 16
NS = CW // SL
NW = 32
EPAD = NW * CW * CH

_mesh = plsc.VectorSubcoreMesh(core_axis_name="c", subcore_axis_name="s")

_no_layout = pltpu.CompilerParams()
if "needs_layout_passes" in pltpu.CompilerParams.__dataclass_fields__:
    _no_layout = dataclasses.replace(_no_layout, needs_layout_passes=False)


@functools.partial(
    pl.kernel,
    out_type=[
        jax.ShapeDtypeStruct((2, 16, NPAD), jnp.float32),
        jax.ShapeDtypeStruct((2, 16, NPAD), jnp.float32),
    ],
    mesh=_mesh,
    scratch_types=[
        pltpu.VMEM((CW, CH), jnp.int32),
        pltpu.VMEM((CW, CH), jnp.int32),
        pltpu.VMEM((NPAD,), jnp.float32),
        pltpu.VMEM((NPAD,), jnp.float32),
    ],
    compiler_params=_no_layout,
)
def _deg_kernel(src_hbm, dst_hbm, dsrc_out, ddst_out, sbuf, dbuf, hs, hd):
    c = lax.axis_index("c")
    s = lax.axis_index("s")
    w = c * 16 + s
    pltpu.sync_copy(src_hbm.at[pl.ds(w * CW, CW)], sbuf)
    pltpu.sync_copy(dst_hbm.at[pl.ds(w * CW, CW)], dbuf)
    zeros = jnp.zeros((16,), jnp.float32)

    @pl.loop(0, NPAD // 16)
    def _(i):
        hs[pl.ds(i * 16, 16)] = zeros
        hd[pl.ds(i * 16, 16)] = zeros

    ones = jnp.ones((16,), jnp.float32)

    @pl.loop(0, CW)
    def _(j):
        for k in range(CH // 16):
            plsc.addupdate_scatter(hs, [sbuf[j, pl.ds(k * 16, 16)]], ones)
            plsc.addupdate_scatter(hd, [dbuf[j, pl.ds(k * 16, 16)]], ones)

    pltpu.sync_copy(hs, dsrc_out.at[c, s])
    pltpu.sync_copy(hd, ddst_out.at[c, s])


@functools.partial(
    pl.kernel,
    out_type=jax.ShapeDtypeStruct((2, NPAD, D), jnp.float32),
    mesh=_mesh,
    scratch_types=[
        pltpu.VMEM((2, SL, CH), jnp.int32),
        pltpu.VMEM((2, SL, CH), jnp.int32),
        pltpu.VMEM((CH, D), jnp.float32),
        pltpu.VMEM((CH, D), jnp.float32),
        pltpu.VMEM_SHARED((NPAD, D), jnp.float32),
        pltpu.SemaphoreType.DMA,
        pltpu.SemaphoreType.DMA,
        pltpu.SemaphoreType.DMA,
    ],
)
def _agg_kernel(h_hbm, src_hbm, dst_hbm, zeros_hbm,
                acc_out, sidx, didx, r0, r1, acc_sh, sem0, sem1, isem):
    c = lax.axis_index("c")
    s = lax.axis_index("s")
    w = c * 16 + s
    base = w * CW
    pltpu.sync_copy(zeros_hbm, acc_sh.at[pl.ds(s * RPS, RPS)])
    pltpu.sync_copy(src_hbm.at[pl.ds(base, SL)], sidx.at[0])
    pltpu.sync_copy(dst_hbm.at[pl.ds(base, SL)], didx.at[0])
    plsc.subcore_barrier()

    pltpu.async_copy(h_hbm.at[sidx.at[0, 0]], r0, sem0)

    @pl.loop(0, NS)
    def _(rs):
        rb = lax.rem(rs, 2)
        nxt = base + (rs + 1) * SL

        @pl.when(rs + 1 < NS)
        def _():
            pltpu.async_copy(src_hbm.at[pl.ds(nxt, SL)], sidx.at[1 - rb], isem)
            pltpu.async_copy(dst_hbm.at[pl.ds(nxt, SL)], didx.at[1 - rb], isem)

        @pl.loop(0, SL // 2)
        def _(t):
            g = t * 2
            pltpu.make_async_copy(h_hbm.at[sidx.at[rb, g]], r0, sem0).wait()
            pltpu.async_copy(h_hbm.at[sidx.at[rb, g + 1]], r1, sem1)
            pltpu.sync_copy(r0, acc_sh.at[didx.at[rb, g]], add=True)
            pltpu.make_async_copy(h_hbm.at[sidx.at[rb, g + 1]], r1, sem1).wait()

            @pl.when(g + 2 < SL)
            def _():
                pltpu.async_copy(h_hbm.at[sidx.at[rb, g + 2]], r0, sem0)

            pltpu.sync_copy(r1, acc_sh.at[didx.at[rb, g + 1]], add=True)

        @pl.when(rs + 1 < NS)
        def _():
            pltpu.make_async_copy(src_hbm.at[pl.ds(nxt, SL)],
                                  sidx.at[1 - rb], isem).wait()
            pltpu.make_async_copy(dst_hbm.at[pl.ds(nxt, SL)],
                                  didx.at[1 - rb], isem).wait()
            pltpu.async_copy(h_hbm.at[sidx.at[1 - rb, 0]], r0, sem0)

    plsc.subcore_barrier()
    pltpu.sync_copy(acc_sh.at[pl.ds(s * RPS, RPS)],
                    acc_out.at[c, pl.ds(s * RPS, RPS)])


def _prep_body(feat_ref, dsrc_ref, ddst_ref, h1_ref, no_ref, ni_ref):
    dsrc = jnp.sum(dsrc_ref[...], axis=(0, 1))[:, None]
    ddst = jnp.sum(ddst_ref[...], axis=(0, 1))[:, None]
    no = jnp.where(dsrc > 0, lax.rsqrt(jnp.maximum(dsrc, 1.0)), 0.0)
    ni = jnp.where(ddst > 0, lax.rsqrt(jnp.maximum(ddst, 1.0)), 0.0)
    no_ref[...] = no
    ni_ref[...] = ni
    h1_ref[...] = feat_ref[...] * no


def _mid_body(acc_ref, ni_ref, no_ref, w_ref, b_ref, out_ref):
    agg = (acc_ref[0] + acc_ref[1]) * ni_ref[...]
    o = jnp.dot(agg, w_ref[...], preferred_element_type=jnp.float32) + b_ref[...]
    out_ref[...] = jnp.maximum(o, 0.0) * no_ref[...]


def _final_body(acc_ref, ni_ref, w_ref, b_ref, out_ref):
    agg = (acc_ref[0] + acc_ref[1]) * ni_ref[...]
    out_ref[...] = (
        jnp.dot(agg, w_ref[...], preferred_element_type=jnp.float32) + b_ref[...]
    )


_BP = 2048
_BF = 2000

_prep_call = pl.pallas_call(
    _prep_body,
    grid=(NPAD // _BP,),
    in_specs=[
        pl.BlockSpec((_BP, D), lambda i: (i, 0)),
        pl.BlockSpec((2, 16, _BP), lambda i: (0, 0, i)),
        pl.BlockSpec((2, 16, _BP), lambda i: (0, 0, i)),
    ],
    out_specs=[
        pl.BlockSpec((_BP, D), lambda i: (i, 0)),
        pl.BlockSpec((_BP, 1), lambda i: (i, 0)),
        pl.BlockSpec((_BP, 1), lambda i: (i, 0)),
    ],
    out_shape=[
        jax.ShapeDtypeStruct((NPAD, D), jnp.float32),
        jax.ShapeDtypeStruct((NPAD, 1), jnp.float32),
        jax.ShapeDtypeStruct((NPAD, 1), jnp.float32),
    ],
)

_mid_call = pl.pallas_call(
    _mid_body,
    grid=(NPAD // _BP,),
    in_specs=[
        pl.BlockSpec((2, _BP, D), lambda i: (0, i, 0)),
        pl.BlockSpec((_BP, 1), lambda i: (i, 0)),
        pl.BlockSpec((_BP, 1), lambda i: (i, 0)),
        pl.BlockSpec((D, D), lambda i: (0, 0)),
        pl.BlockSpec((1, D), lambda i: (0, 0)),
    ],
    out_specs=pl.BlockSpec((_BP, D), lambda i: (i, 0)),
    out_shape=jax.ShapeDtypeStruct((NPAD, D), jnp.float32),
)

_final_call = pl.pallas_call(
    _final_body,
    grid=(N // _BF,),
    in_specs=[
        pl.BlockSpec((2, _BF, D), lambda i: (0, i, 0)),
        pl.BlockSpec((_BF, 1), lambda i: (i, 0)),
        pl.BlockSpec((D, D), lambda i: (0, 0)),
        pl.BlockSpec((1, D), lambda i: (0, 0)),
    ],
    out_specs=pl.BlockSpec((_BF, D), lambda i: (i, 0)),
    out_shape=jax.ShapeDtypeStruct((N, D), jnp.float32),
)


def kernel(features, edge_index, W1, b1, W2, b2):
    edge_index = edge_index.astype(jnp.int32)
    pad = jnp.full((EPAD - E,), N, jnp.int32)
    src2d = jnp.concatenate([edge_index[0], pad]).reshape(NW * CW, CH)
    dst2d = jnp.concatenate([edge_index[1], pad]).reshape(NW * CW, CH)
    feat_pad = jnp.pad(features, ((0, NPAD - N), (0, 0)))
    zeros128 = jnp.zeros((RPS, D), jnp.float32)

    dsrc_p, ddst_p = _deg_kernel(src2d, dst2d)
    h1, norm_out, norm_in = _prep_call(feat_pad, dsrc_p, ddst_p)
    acc1 = _agg_kernel(h1, src2d, dst2d, zeros128)
    h2 = _mid_call(acc1, norm_in, norm_out, W1, b1.reshape(1, D))
    acc2 = _agg_kernel(h2, src2d, dst2d, zeros128)
    return _final_call(acc2, norm_in, W2, b2.reshape(1, D))
